# Initial kernel scaffold; baseline (speedup 1.0000x reference)
import jax
import jax.numpy as jnp
from jax.experimental import pallas as pl


def kernel(feature, edge_index, e, W, b):
    raise NotImplementedError("write your pallas kernel here")


# speedup vs baseline: 4.1817x; 4.1817x over previous
import functools

import jax
import jax.numpy as jnp
from jax import lax
from jax.experimental import pallas as pl
from jax.experimental.pallas import tpu as pltpu
from jax.experimental.pallas import tpu_sc as plsc

N_NODES = 10000
N_EDGES = 320000
D = 128

NC = 2
NS = 16
NW = NC * NS
EDGES_PER_W = N_EDGES // NW
CHUNK = 80
NCHUNK = EDGES_PER_W // CHUNK
ROWS_MAIN = 624
ROWS_TAIL = N_NODES - NS * ROWS_MAIN


def _sc_body(feat_hbm, src_hbm, dst_hbm, agg_out, cnt_out,
             src_v, dst_v, rows_v, acc, sem):
    c = lax.axis_index("c")
    s = lax.axis_index("s")
    wid = s * NC + c
    row0 = s * ROWS_MAIN
    ebase0 = wid * EDGES_PER_W
    nfull = ROWS_MAIN // CHUNK
    rem = ROWS_MAIN - nfull * CHUNK

    def _fill_rows(val16):
        def _frow(r, carry):
            for cc in range(D // 16):
                rows_v[r, pl.ds(cc * 16, 16)] = val16
            return carry
        lax.fori_loop(0, CHUNK, _frow, 0)

    def _zero_acc():
        for k in range(nfull):
            pltpu.sync_copy(rows_v, acc.at[pl.ds(row0 + k * CHUNK, CHUNK)])
        pltpu.sync_copy(rows_v.at[pl.ds(0, rem)],
                        acc.at[pl.ds(row0 + nfull * CHUNK, rem)])

        @pl.when(s == NS - 1)
        def _zero_tail():
            pltpu.sync_copy(rows_v.at[pl.ds(0, ROWS_TAIL)],
                            acc.at[pl.ds(NS * ROWS_MAIN, ROWS_TAIL)])

    def _dump_acc(out):
        pltpu.sync_copy(acc.at[pl.ds(row0, ROWS_MAIN)],
                        out.at[c, pl.ds(row0, ROWS_MAIN)])

        @pl.when(s == NS - 1)
        def _dump_tail():
            pltpu.sync_copy(acc.at[pl.ds(NS * ROWS_MAIN, ROWS_TAIL)],
                            out.at[c, pl.ds(NS * ROWS_MAIN, ROWS_TAIL)])

    _fill_rows(jnp.zeros((16,), jnp.float32))
    _zero_acc()
    plsc.subcore_barrier()

    def _chunk1(ch, carry):
        ebase = ebase0 + ch * CHUNK
        pltpu.sync_copy(src_hbm.at[pl.ds(ebase, CHUNK)], src_v)
        pltpu.sync_copy(dst_hbm.at[pl.ds(ebase, CHUNK)], dst_v)
        pltpu.async_copy(feat_hbm.at[src_v], rows_v, sem).wait()
        pltpu.sync_copy(rows_v, acc.at[dst_v], add=True)
        return carry
    lax.fori_loop(0, NCHUNK, _chunk1, 0)

    plsc.subcore_barrier()
    _dump_acc(agg_out)
    plsc.subcore_barrier()

    _fill_rows(jnp.zeros((16,), jnp.float32))
    _zero_acc()
    plsc.subcore_barrier()
    _fill_rows(jnp.ones((16,), jnp.float32))

    def _chunk2(ch, carry):
        ebase = ebase0 + ch * CHUNK
        pltpu.sync_copy(dst_hbm.at[pl.ds(ebase, CHUNK)], dst_v)
        pltpu.sync_copy(rows_v, acc.at[dst_v], add=True)
        return carry
    lax.fori_loop(0, NCHUNK, _chunk2, 0)

    plsc.subcore_barrier()
    _dump_acc(cnt_out)


@jax.jit
def _sc_segment(feature, src, dst):
    mesh = plsc.VectorSubcoreMesh(core_axis_name="c", subcore_axis_name="s")
    kfn = pl.kernel(
        _sc_body,
        out_type=[
            jax.ShapeDtypeStruct((NC, N_NODES, D), jnp.float32),
            jax.ShapeDtypeStruct((NC, N_NODES, D), jnp.float32),
        ],
        mesh=mesh,
        scratch_types=[
            pltpu.VMEM((CHUNK,), jnp.int32),
            pltpu.VMEM((CHUNK,), jnp.int32),
            pltpu.VMEM((CHUNK, D), jnp.float32),
            pltpu.VMEM_SHARED((N_NODES, D), jnp.float32),
            pltpu.SemaphoreType.DMA,
        ],
    )
    return kfn(feature, src, dst)


def _tc_finalize_body(feat_ref, agg_ref, cnt_ref, w_ref, b_ref, out_ref):
    agg = agg_ref[0] + agg_ref[1]
    cnt = jnp.maximum(cnt_ref[0, :, 0] + cnt_ref[1, :, 0], 1.0)
    mean = agg / cnt[:, None]
    h = lax.dot_general(mean, w_ref[...], (((1,), (1,)), ((), ())),
                        preferred_element_type=jnp.float32,
                        precision=lax.Precision.HIGHEST)
    out_ref[...] = feat_ref[...] + h + b_ref[...]


@jax.jit
def _tc_finalize(feature, agg2, cnt2, W, b2d):
    R = 1000
    grid = (N_NODES // R,)
    return pl.pallas_call(
        _tc_finalize_body,
        grid=grid,
        in_specs=[
            pl.BlockSpec((R, D), lambda i: (i, 0)),
            pl.BlockSpec((NC, R, D), lambda i: (0, i, 0)),
            pl.BlockSpec((NC, R, D), lambda i: (0, i, 0)),
            pl.BlockSpec((D, D), lambda i: (0, 0)),
            pl.BlockSpec((1, D), lambda i: (0, 0)),
        ],
        out_specs=pl.BlockSpec((R, D), lambda i: (i, 0)),
        out_shape=jax.ShapeDtypeStruct((N_NODES, D), jnp.float32),
    )(feature, agg2, cnt2, W, b2d)


def kernel(feature, edge_index, e, W, b):
    agg2, cnt2 = _sc_segment(feature, edge_index[0], edge_index[1])
    h = _tc_finalize(feature, agg2, cnt2, W, b.reshape(1, D))
    return h, e
